# Initial kernel scaffold; baseline (speedup 1.0000x reference)
import jax
import jax.numpy as jnp
from jax.experimental import pallas as pl


def kernel(x, f, p_idx, o_idx, nw_w, nw_b, hw_w, hw_b):
    raise NotImplementedError("write your pallas kernel here")


# speedup vs baseline: 7.5929x; 7.5929x over previous
import functools

import jax
import jax.numpy as jnp
from jax import lax
from jax.experimental import pallas as pl
from jax.experimental.pallas import tpu as pltpu
from jax.experimental.pallas import tpu_sc as plsc

EMB = 512
NEG_SLOPE = 0.2


@functools.lru_cache(maxsize=None)
def _make_sc_gather(V, D, B):
    info = plsc.get_sparse_core_info()
    NC, NS = info.num_cores, info.num_subcores
    NW = NC * NS
    assert B % (8 * NW) == 0
    b_per_w = B // NW
    CH = 128
    assert b_per_w % CH == 0
    n_ch = b_per_w // CH
    mesh = plsc.VectorSubcoreMesh(core_axis_name="c", subcore_axis_name="s")

    @functools.partial(
        pl.kernel,
        out_type=jax.ShapeDtypeStruct((B, D), jnp.float32),
        mesh=mesh,
        scratch_types=[
            pltpu.VMEM((b_per_w,), jnp.int32),
            pltpu.VMEM((CH, D), jnp.float32),
            pltpu.SemaphoreType.DMA,
        ],
    )
    def gather_kernel(table_hbm, idx_hbm, out_hbm, idx_v, rows_v, sem):
        wid = lax.axis_index("s") * NC + lax.axis_index("c")
        base = wid * b_per_w
        pltpu.sync_copy(idx_hbm.at[pl.ds(base, b_per_w)], idx_v)

        def body(ci, carry):
            pltpu.async_copy(
                table_hbm.at[idx_v.at[pl.ds(ci * CH, CH)]], rows_v, sem
            ).wait()
            pltpu.sync_copy(rows_v, out_hbm.at[pl.ds(base + ci * CH, CH)])
            return carry

        lax.fori_loop(0, n_ch, body, 0)

    return gather_kernel


def _mlp_body(x_ref, h_ref, hf_ref, nf_ref, w0_ref, w1_ref, w2_ref, nwb_ref,
              hw0_ref, hw1_ref, hwb_ref, out_ref, acc_ref, *, nb):
    i = pl.program_id(0)
    h = h_ref[pl.ds(x_ref[0] % 8, 1), :]
    dn = (((1,), (1,)), ((), ()))
    c = lax.dot_general(h, w0_ref[...], dn,
                        preferred_element_type=jnp.float32) + nwb_ref[...]
    z = lax.dot_general(hf_ref[...], w1_ref[...], dn,
                        preferred_element_type=jnp.float32)
    z = z + lax.dot_general(nf_ref[...], w2_ref[...], dn,
                            preferred_element_type=jnp.float32)
    z = z + c
    zl = jnp.where(z >= 0, z, NEG_SLOPE * z)
    part = jnp.sum(zl, axis=0, keepdims=True)

    @pl.when(i == 0)
    def _():
        acc_ref[...] = part

    @pl.when(i > 0)
    def _():
        acc_ref[...] = acc_ref[...] + part

    @pl.when(i == nb - 1)
    def _():
        nbf = acc_ref[...]
        r = lax.dot_general(h, hw0_ref[...], dn,
                            preferred_element_type=jnp.float32)
        r = r + lax.dot_general(nbf, hw1_ref[...], dn,
                                preferred_element_type=jnp.float32)
        r = r + hwb_ref[...]
        out_ref[...] = jnp.where(r >= 0, r, NEG_SLOPE * r)


def _mlp_sum(xarr, f, g, w0, w1, w2, nwb, hw0, hw1, hwb, *, e, be):
    nb = e // be
    grid_spec = pltpu.PrefetchScalarGridSpec(
        num_scalar_prefetch=1,
        grid=(nb,),
        in_specs=[
            pl.BlockSpec((8, EMB), lambda i, xr: (xr[0] // 8, 0)),
            pl.BlockSpec((be, EMB), lambda i, xr: (i, 0)),
            pl.BlockSpec((be, EMB), lambda i, xr: (nb + i, 0)),
            pl.BlockSpec((EMB, EMB), lambda i, xr: (0, 0)),
            pl.BlockSpec((EMB, EMB), lambda i, xr: (0, 0)),
            pl.BlockSpec((EMB, EMB), lambda i, xr: (0, 0)),
            pl.BlockSpec((1, EMB), lambda i, xr: (0, 0)),
            pl.BlockSpec((EMB, EMB), lambda i, xr: (0, 0)),
            pl.BlockSpec((EMB, EMB), lambda i, xr: (0, 0)),
            pl.BlockSpec((1, EMB), lambda i, xr: (0, 0)),
        ],
        out_specs=pl.BlockSpec((1, EMB), lambda i, xr: (0, 0)),
        scratch_shapes=[pltpu.VMEM((1, EMB), jnp.float32)],
    )
    return pl.pallas_call(
        functools.partial(_mlp_body, nb=nb),
        grid_spec=grid_spec,
        out_shape=jax.ShapeDtypeStruct((1, EMB), jnp.float32),
        compiler_params=pltpu.CompilerParams(
            dimension_semantics=("arbitrary",),
        ),
    )(xarr, f, g, g, w0, w1, w2, nwb, hw0, hw1, hwb)


def kernel(x, f, p_idx, o_idx, nw_w, nw_b, hw_w, hw_b):
    e = p_idx.shape[0]
    idx_all = jnp.concatenate(
        [p_idx.astype(jnp.int32), o_idx.astype(jnp.int32)], axis=0)
    g = _make_sc_gather(f.shape[0], EMB, 2 * e)(f, idx_all)

    xarr = jnp.reshape(x, (1,)).astype(jnp.int32)
    w0 = nw_w[:, :EMB]
    w1 = nw_w[:, EMB:2 * EMB]
    w2 = nw_w[:, 2 * EMB:]
    hw0 = hw_w[:, :EMB]
    hw1 = hw_w[:, EMB:]
    nwb = jnp.reshape(nw_b, (1, EMB))
    hwb = jnp.reshape(hw_b, (1, EMB))
    return _mlp_sum(xarr, f, g, w0, w1, w2, nwb, hw0, hw1, hwb,
                    e=e, be=2048)


# speedup vs baseline: 8.0053x; 1.0543x over previous
import functools

import jax
import jax.numpy as jnp
from jax import lax
from jax.experimental import pallas as pl
from jax.experimental.pallas import tpu as pltpu
from jax.experimental.pallas import tpu_sc as plsc

EMB = 512
NEG_SLOPE = 0.2


@functools.lru_cache(maxsize=None)
def _make_sc_gather(V, D, B):
    info = plsc.get_sparse_core_info()
    NC, NS = info.num_cores, info.num_subcores
    NW = NC * NS
    assert B % (8 * NW) == 0
    b_per_w = B // NW
    CH = 64
    assert b_per_w % (2 * CH) == 0
    n_ch = b_per_w // CH
    mesh = plsc.VectorSubcoreMesh(core_axis_name="c", subcore_axis_name="s")

    @functools.partial(
        pl.kernel,
        out_type=jax.ShapeDtypeStruct((B, D), jnp.float32),
        mesh=mesh,
        scratch_types=[
            pltpu.VMEM((b_per_w,), jnp.int32),
            pltpu.VMEM((CH, D), jnp.float32),
            pltpu.VMEM((CH, D), jnp.float32),
            pltpu.SemaphoreType.DMA,
            pltpu.SemaphoreType.DMA,
        ],
    )
    def gather_kernel(table_hbm, idx_hbm, out_hbm, idx_v, rows0, rows1,
                      sem0, sem1):
        wid = lax.axis_index("s") * NC + lax.axis_index("c")
        base = wid * b_per_w
        bufs = (rows0, rows1)
        sems = (sem0, sem1)
        pltpu.sync_copy(idx_hbm.at[pl.ds(base, b_per_w)], idx_v)

        def start(ci, b):
            pltpu.make_async_copy(
                table_hbm.at[idx_v.at[pl.ds(ci * CH, CH)]], bufs[b], sems[b]
            ).start()

        start(0, 0)
        start(1, 1)

        def body(g, carry):
            for b in range(2):
                ci = g + b
                pltpu.make_async_copy(
                    table_hbm.at[idx_v.at[pl.ds(ci * CH, CH)]],
                    bufs[b], sems[b]
                ).wait()
                pltpu.sync_copy(bufs[b], out_hbm.at[pl.ds(base + ci * CH, CH)])

                @pl.when(ci + 2 < n_ch)
                def _():
                    start(ci + 2, b)
            return carry

        lax.fori_loop(0, n_ch // 2, lambda g, c: body(g * 2, c), 0)

    return gather_kernel


def _mlp_body(x_ref, h_ref, hf_ref, nf_ref, w0_ref, w1_ref, w2_ref, nwb_ref,
              hw0_ref, hw1_ref, hwb_ref, out_ref, acc_ref, *, nb):
    i = pl.program_id(0)
    h = h_ref[pl.ds(x_ref[0] % 8, 1), :]
    dn = (((1,), (1,)), ((), ()))
    c = lax.dot_general(h, w0_ref[...], dn,
                        preferred_element_type=jnp.float32) + nwb_ref[...]
    z = lax.dot_general(hf_ref[...].astype(jnp.bfloat16), w1_ref[...], dn,
                        preferred_element_type=jnp.float32)
    z = z + lax.dot_general(nf_ref[...].astype(jnp.bfloat16), w2_ref[...], dn,
                            preferred_element_type=jnp.float32)
    z = z + c
    zl = jnp.where(z >= 0, z, NEG_SLOPE * z)
    part = jnp.sum(zl, axis=0, keepdims=True)

    @pl.when(i == 0)
    def _():
        acc_ref[...] = part

    @pl.when(i > 0)
    def _():
        acc_ref[...] = acc_ref[...] + part

    @pl.when(i == nb - 1)
    def _():
        nbf = acc_ref[...]
        r = lax.dot_general(h, hw0_ref[...], dn,
                            preferred_element_type=jnp.float32)
        r = r + lax.dot_general(nbf, hw1_ref[...], dn,
                                preferred_element_type=jnp.float32)
        r = r + hwb_ref[...]
        out_ref[...] = jnp.where(r >= 0, r, NEG_SLOPE * r)


def _mlp_sum(xarr, f, g, w0, w1, w2, nwb, hw0, hw1, hwb, *, e, be):
    nb = e // be
    grid_spec = pltpu.PrefetchScalarGridSpec(
        num_scalar_prefetch=1,
        grid=(nb,),
        in_specs=[
            pl.BlockSpec((8, EMB), lambda i, xr: (xr[0] // 8, 0)),
            pl.BlockSpec((be, EMB), lambda i, xr: (i, 0)),
            pl.BlockSpec((be, EMB), lambda i, xr: (nb + i, 0)),
            pl.BlockSpec((EMB, EMB), lambda i, xr: (0, 0)),
            pl.BlockSpec((EMB, EMB), lambda i, xr: (0, 0)),
            pl.BlockSpec((EMB, EMB), lambda i, xr: (0, 0)),
            pl.BlockSpec((1, EMB), lambda i, xr: (0, 0)),
            pl.BlockSpec((EMB, EMB), lambda i, xr: (0, 0)),
            pl.BlockSpec((EMB, EMB), lambda i, xr: (0, 0)),
            pl.BlockSpec((1, EMB), lambda i, xr: (0, 0)),
        ],
        out_specs=pl.BlockSpec((1, EMB), lambda i, xr: (0, 0)),
        scratch_shapes=[pltpu.VMEM((1, EMB), jnp.float32)],
    )
    return pl.pallas_call(
        functools.partial(_mlp_body, nb=nb),
        grid_spec=grid_spec,
        out_shape=jax.ShapeDtypeStruct((1, EMB), jnp.float32),
        compiler_params=pltpu.CompilerParams(
            dimension_semantics=("arbitrary",),
        ),
    )(xarr, f, g, g, w0, w1, w2, nwb, hw0, hw1, hwb)


def kernel(x, f, p_idx, o_idx, nw_w, nw_b, hw_w, hw_b):
    e = p_idx.shape[0]
    idx_all = jnp.concatenate(
        [p_idx.astype(jnp.int32), o_idx.astype(jnp.int32)], axis=0)
    g = _make_sc_gather(f.shape[0], EMB, 2 * e)(f, idx_all)

    xarr = jnp.reshape(x, (1,)).astype(jnp.int32)
    w0 = nw_w[:, :EMB]
    w1 = nw_w[:, EMB:2 * EMB].astype(jnp.bfloat16)
    w2 = nw_w[:, 2 * EMB:].astype(jnp.bfloat16)
    hw0 = hw_w[:, :EMB]
    hw1 = hw_w[:, EMB:]
    nwb = jnp.reshape(nw_b, (1, EMB))
    hwb = jnp.reshape(hw_b, (1, EMB))
    return _mlp_sum(xarr, f, g, w0, w1, w2, nwb, hw0, hw1, hwb,
                    e=e, be=2048)
